# Initial kernel scaffold; baseline (speedup 1.0000x reference)
import functools

import numpy as np
import jax
import jax.numpy as jnp
from jax import lax
from jax.experimental import pallas as pl
from jax.experimental.pallas import tpu as pltpu
from jax.experimental.pallas import tpu_sc as plsc

_POS_DIM = 3
_N_LEVELS = 16
_N_FEATS = 2
_CAP = 2 ** 18
_N = 262144
_OUT = 64

_SFC = [1.0 / float(np.sqrt((i + 1.0) * (i + 2.0))) for i in range(_POS_DIM)]
_PRIMES_I32 = [int(np.int32(np.uint32(p)))
               for p in (2654435761, 805459861, 3674653429)]


def _scales_np():
    l = np.arange(_N_LEVELS, dtype=np.float64)
    g = (1000.0 / 10.0) ** (l / max(_N_LEVELS - 1, 1))
    return (10.0 * g * 1.0).astype(np.float32)


_info = plsc.get_sparse_core_info()
_NC, _NS = _info.num_cores, _info.num_subcores
_NW = _NC * _NS
_C = 1024
_PPW = _N // _NW
_NCHUNK = _PPW // _C
_G = _C // 16


def _sc_encode(xT, tab, lvc):
    mesh = plsc.VectorSubcoreMesh(core_axis_name="c", subcore_axis_name="s")

    @functools.partial(
        pl.kernel,
        out_type=jax.ShapeDtypeStruct((2 * _N_LEVELS, _N), jnp.float32),
        mesh=mesh,
        scratch_types=[
            pltpu.VMEM((_POS_DIM, _C), jnp.float32),
            pltpu.VMEM((4 * _C,), jnp.int32),
            pltpu.VMEM((4 * _C,), jnp.float32),
            pltpu.VMEM((4 * _C, _N_FEATS), jnp.float32),
            pltpu.VMEM((2 * _N_LEVELS, _C), jnp.float32),
            pltpu.VMEM((8 * _N_LEVELS,), jnp.float32),
            pltpu.SemaphoreType.DMA,
        ],
    )
    def enc_kernel(xT_hbm, tab_hbm, lvc_hbm, out_hbm,
                   x_v, idx_v, bary_v, feats_v, enc_v, lvc_v, sem):
        wid = lax.axis_index("s") * _NC + lax.axis_index("c")
        pltpu.sync_copy(lvc_hbm, lvc_v)
        iota16 = lax.iota(jnp.int32, 16)
        zeros16 = jnp.zeros((16,), jnp.int32)

        def chunk_body(ci, carry):
            base = wid * _PPW + ci * _C
            pltpu.sync_copy(xT_hbm.at[:, pl.ds(base, _C)], x_v)

            def level_body(l, carry2):
                lb = 8 * l
                sc_v = plsc.load_gather(lvc_v, [zeros16 + lb])
                sh0 = plsc.load_gather(lvc_v, [zeros16 + (lb + 1)])
                sh1 = plsc.load_gather(lvc_v, [zeros16 + (lb + 2)])
                sh2 = plsc.load_gather(lvc_v, [zeros16 + (lb + 3)])
                lvl_off = l * _CAP

                def group_body(g, carry3):
                    o = g * 16
                    x0 = x_v[0, pl.ds(o, 16)]
                    x1 = x_v[1, pl.ds(o, 16)]
                    x2 = x_v[2, pl.ds(o, 16)]
                    cf0 = (x0 * sc_v + sh0) * _SFC[0]
                    cf1 = (x1 * sc_v + sh1) * _SFC[1]
                    cf2 = (x2 * sc_v + sh2) * _SFC[2]
                    s1 = cf1 + cf2
                    e0 = cf0 + s1
                    e1 = s1 - cf0
                    e2 = cf2 - 2.0 * cf1
                    e3 = -3.0 * cf2
                    ee = [e0, e1, e2, e3]
                    rr = []
                    for e in ee:
                        v = e * 0.25
                        vi = v.astype(jnp.int32)
                        vf = vi.astype(jnp.float32)
                        fl = jnp.where(vf > v, vf - 1.0, vf)
                        ce = jnp.where(vf < v, vf + 1.0, vf)
                        up = ce * 4.0
                        down = fl * 4.0
                        rr.append(jnp.where(up - e < e - down, up, down))
                    ssum = (rr[0] + rr[1]) + (rr[2] + rr[3])
                    rsum = (ssum * 0.25).astype(jnp.int32)
                    dd = [ee[i] - rr[i] for i in range(4)]
                    rank = [rsum, rsum, rsum, rsum]
                    for i in range(4):
                        for j in range(i + 1, 4):
                            c = dd[i] < dd[j]
                            rank[i] = rank[i] + jnp.where(c, 1, 0)
                            rank[j] = rank[j] + jnp.where(c, 0, 1)
                    ri = [r.astype(jnp.int32) for r in rr]
                    dl = []
                    for i in range(4):
                        low = rank[i] < 0
                        high = rank[i] > 3
                        ri[i] = jnp.where(low, ri[i] + 4,
                                          jnp.where(high, ri[i] - 4, ri[i]))
                        rank[i] = jnp.where(low, rank[i] + 4,
                                            jnp.where(high, rank[i] - 4,
                                                      rank[i]))
                        dl.append((ee[i] - ri[i].astype(jnp.float32)) * 0.25)
                    gg = []
                    for r in range(4):
                        acc = jnp.where(rank[0] == r, dl[0], 0.0)
                        for i in range(1, 4):
                            acc = acc + jnp.where(rank[i] == r, dl[i], 0.0)
                        gg.append(acc)
                    bary = [1.0 + gg[3] - gg[0], gg[2] - gg[3],
                            gg[1] - gg[2], gg[0] - gg[1]]
                    for r in range(4):
                        if r == 0:
                            k0, k1, k2 = ri[0], ri[1], ri[2]
                        else:
                            k0 = ri[0] + jnp.where(rank[0] > 3 - r, r - 4, r)
                            k1 = ri[1] + jnp.where(rank[1] > 3 - r, r - 4, r)
                            k2 = ri[2] + jnp.where(rank[2] > 3 - r, r - 4, r)
                        h = ((k0 * _PRIMES_I32[0]) ^ (k1 * _PRIMES_I32[1])
                             ^ (k2 * _PRIMES_I32[2]))
                        idx = (h & (_CAP - 1)) + lvl_off
                        idx_v[pl.ds(r * _C + o, 16)] = idx
                        bary_v[pl.ds(r * _C + o, 16)] = bary[r]
                    return carry3

                lax.fori_loop(0, _G, group_body, 0)
                pltpu.async_copy(tab_hbm.at[idx_v], feats_v, sem).wait()

                def blend_body(g, carry3):
                    o = g * 16
                    acc0 = jnp.zeros((16,), jnp.float32)
                    acc1 = jnp.zeros((16,), jnp.float32)
                    for r in range(4):
                        row = (r * _C + o) + iota16
                        f0 = plsc.load_gather(feats_v, [row, zeros16])
                        f1 = plsc.load_gather(feats_v, [row, zeros16 + 1])
                        br = bary_v[pl.ds(r * _C + o, 16)]
                        acc0 = acc0 + br * f0
                        acc1 = acc1 + br * f1
                    enc_v[2 * l, pl.ds(o, 16)] = acc0
                    enc_v[2 * l + 1, pl.ds(o, 16)] = acc1
                    return carry3

                lax.fori_loop(0, _G, blend_body, 0)
                return carry2

            lax.fori_loop(0, _N_LEVELS, level_body, 0)
            pltpu.sync_copy(enc_v, out_hbm.at[:, pl.ds(base, _C)])
            return carry

        lax.fori_loop(0, _NCHUNK, chunk_body, 0)

    return enc_kernel(xT, tab, lvc)


def _tc_decode(enc, W, b):
    BN = 2048

    def body(enc_ref, w_ref, b_ref, o_ref):
        o_ref[...] = lax.dot_general(
            enc_ref[...], w_ref[...], (((0,), (1,)), ((), ())),
            preferred_element_type=jnp.float32) + b_ref[...]

    return pl.pallas_call(
        body,
        grid=(_N // BN,),
        in_specs=[
            pl.BlockSpec((2 * _N_LEVELS, BN), lambda i: (0, i)),
            pl.BlockSpec((_OUT, 2 * _N_LEVELS), lambda i: (0, 0)),
            pl.BlockSpec((1, _OUT), lambda i: (0, 0)),
        ],
        out_specs=pl.BlockSpec((BN, _OUT), lambda i: (i, 0)),
        out_shape=jax.ShapeDtypeStruct((_N, _OUT), jnp.float32),
    )(enc, W, b[None, :])


def kernel(x, table, shifts, W, b):
    xT = x.T
    tab = table.reshape(_N_LEVELS * _CAP, _N_FEATS)
    scales = jnp.asarray(_scales_np())
    lvc = jnp.concatenate(
        [scales[:, None], shifts, jnp.zeros((_N_LEVELS, 4), jnp.float32)],
        axis=1).reshape(-1)
    enc = _sc_encode(xT, tab, lvc)
    return _tc_decode(enc, W, b)


# speedup vs baseline: 19.0634x; 19.0634x over previous
import functools

import numpy as np
import jax
import jax.numpy as jnp
from jax import lax
from jax.experimental import pallas as pl
from jax.experimental.pallas import tpu as pltpu
from jax.experimental.pallas import tpu_sc as plsc

_POS_DIM = 3
_N_LEVELS = 16
_N_FEATS = 2
_CAP = 2 ** 18
_N = 262144
_OUT = 64

_SFC = [1.0 / float(np.sqrt((i + 1.0) * (i + 2.0))) for i in range(_POS_DIM)]
_PRIMES_I32 = [int(np.int32(np.uint32(p)))
               for p in (2654435761, 805459861, 3674653429)]


def _scales_np():
    l = np.arange(_N_LEVELS, dtype=np.float64)
    g = (1000.0 / 10.0) ** (l / max(_N_LEVELS - 1, 1))
    return (10.0 * g * 1.0).astype(np.float32)


_NC, _NS = 2, 16
_NW = _NC * _NS
_C = 1024
_PPW = _N // _NW
_NCHUNK = _PPW // _C
_G = _C // 16


def _sc_encode(xT, tab, lvc):
    mesh = plsc.VectorSubcoreMesh(
        core_axis_name="c", subcore_axis_name="s",
        num_cores=_NC, num_subcores=_NS)

    @functools.partial(
        pl.kernel,
        out_type=jax.ShapeDtypeStruct((2 * _N_LEVELS, _N), jnp.float32),
        mesh=mesh,
        compiler_params=pltpu.CompilerParams(
            needs_layout_passes=False, use_tc_tiling_on_sc=False),
        scratch_types=[
            pltpu.VMEM((_POS_DIM, _C), jnp.float32),
            pltpu.VMEM((4 * _C,), jnp.int32),
            pltpu.VMEM((4 * _C,), jnp.int32),
            pltpu.VMEM((4 * _C,), jnp.float32),
            pltpu.VMEM((4 * _C,), jnp.float32),
            pltpu.VMEM((4 * _C,), jnp.float32),
            pltpu.VMEM((2 * _N_LEVELS, _C), jnp.float32),
            pltpu.VMEM((8 * _N_LEVELS,), jnp.float32),
            pltpu.SemaphoreType.DMA,
            pltpu.SemaphoreType.DMA,
        ],
    )
    def enc_kernel(xT_hbm, tab_hbm, lvc_hbm, out_hbm,
                   x_v, idx0_v, idx1_v, bary_v, f0_v, f1_v, enc_v, lvc_v,
                   sem0, sem1):
        wid = lax.axis_index("s") * _NC + lax.axis_index("c")
        pltpu.sync_copy(lvc_hbm, lvc_v)
        iota16 = lax.iota(jnp.int32, 16)
        zeros16 = jnp.zeros((16,), jnp.int32)

        def chunk_body(ci, carry):
            base = wid * _PPW + ci * _C
            pltpu.sync_copy(xT_hbm.at[:, pl.ds(base, _C)], x_v)

            def level_body(l, carry2):
                lb = 8 * l
                sc_v = plsc.load_gather(lvc_v, [zeros16 + lb])
                sh0 = plsc.load_gather(lvc_v, [zeros16 + (lb + 1)])
                sh1 = plsc.load_gather(lvc_v, [zeros16 + (lb + 2)])
                sh2 = plsc.load_gather(lvc_v, [zeros16 + (lb + 3)])
                lvl_off = l * _CAP

                def group_body(g, carry3):
                    o = g * 16
                    x0 = x_v[0, pl.ds(o, 16)]
                    x1 = x_v[1, pl.ds(o, 16)]
                    x2 = x_v[2, pl.ds(o, 16)]
                    cf0 = (x0 * sc_v + sh0) * _SFC[0]
                    cf1 = (x1 * sc_v + sh1) * _SFC[1]
                    cf2 = (x2 * sc_v + sh2) * _SFC[2]
                    s1 = cf1 + cf2
                    e0 = cf0 + s1
                    e1 = s1 - cf0
                    e2 = cf2 - 2.0 * cf1
                    e3 = -3.0 * cf2
                    ee = [e0, e1, e2, e3]
                    rr = []
                    for e in ee:
                        v = e * 0.25
                        vi = v.astype(jnp.int32)
                        vf = vi.astype(jnp.float32)
                        fl = jnp.where(vf > v, vf - 1.0, vf)
                        ce = jnp.where(vf < v, vf + 1.0, vf)
                        up = ce * 4.0
                        down = fl * 4.0
                        rr.append(jnp.where(up - e < e - down, up, down))
                    ssum = (rr[0] + rr[1]) + (rr[2] + rr[3])
                    rsum = (ssum * 0.25).astype(jnp.int32)
                    dd = [ee[i] - rr[i] for i in range(4)]
                    rank = [rsum, rsum, rsum, rsum]
                    for i in range(4):
                        for j in range(i + 1, 4):
                            c = dd[i] < dd[j]
                            rank[i] = rank[i] + jnp.where(c, 1, 0)
                            rank[j] = rank[j] + jnp.where(c, 0, 1)
                    ri = [r.astype(jnp.int32) for r in rr]
                    dl = []
                    for i in range(4):
                        low = rank[i] < 0
                        high = rank[i] > 3
                        ri[i] = jnp.where(low, ri[i] + 4,
                                          jnp.where(high, ri[i] - 4, ri[i]))
                        rank[i] = jnp.where(low, rank[i] + 4,
                                            jnp.where(high, rank[i] - 4,
                                                      rank[i]))
                        dl.append((ee[i] - ri[i].astype(jnp.float32)) * 0.25)
                    gg = []
                    for r in range(4):
                        acc = jnp.where(rank[0] == r, dl[0], 0.0)
                        for i in range(1, 4):
                            acc = acc + jnp.where(rank[i] == r, dl[i], 0.0)
                        gg.append(acc)
                    bary = [1.0 + gg[3] - gg[0], gg[2] - gg[3],
                            gg[1] - gg[2], gg[0] - gg[1]]
                    for r in range(4):
                        if r == 0:
                            k0, k1, k2 = ri[0], ri[1], ri[2]
                        else:
                            k0 = ri[0] + jnp.where(rank[0] > 3 - r, r - 4, r)
                            k1 = ri[1] + jnp.where(rank[1] > 3 - r, r - 4, r)
                            k2 = ri[2] + jnp.where(rank[2] > 3 - r, r - 4, r)
                        h = ((k0 * _PRIMES_I32[0]) ^ (k1 * _PRIMES_I32[1])
                             ^ (k2 * _PRIMES_I32[2]))
                        fpos = ((h & (_CAP - 1)) + lvl_off) * 2
                        idx0_v[pl.ds(r * _C + o, 16)] = fpos
                        idx1_v[pl.ds(r * _C + o, 16)] = fpos + 1
                        bary_v[pl.ds(r * _C + o, 16)] = bary[r]
                    return carry3

                lax.fori_loop(0, _G, group_body, 0)
                cp0 = pltpu.async_copy(tab_hbm.at[idx0_v], f0_v, sem0)
                cp1 = pltpu.async_copy(tab_hbm.at[idx1_v], f1_v, sem1)
                cp0.wait()
                cp1.wait()

                def blend_body(g, carry3):
                    o = g * 16
                    acc0 = jnp.zeros((16,), jnp.float32)
                    acc1 = jnp.zeros((16,), jnp.float32)
                    for r in range(4):
                        f0 = f0_v[pl.ds(r * _C + o, 16)]
                        f1 = f1_v[pl.ds(r * _C + o, 16)]
                        br = bary_v[pl.ds(r * _C + o, 16)]
                        acc0 = acc0 + br * f0
                        acc1 = acc1 + br * f1
                    enc_v[2 * l, pl.ds(o, 16)] = acc0
                    enc_v[2 * l + 1, pl.ds(o, 16)] = acc1
                    return carry3

                lax.fori_loop(0, _G, blend_body, 0)
                return carry2

            lax.fori_loop(0, _N_LEVELS, level_body, 0)
            pltpu.sync_copy(enc_v, out_hbm.at[:, pl.ds(base, _C)])
            return carry

        lax.fori_loop(0, _NCHUNK, chunk_body, 0)

    return enc_kernel(xT, tab, lvc)


def _tc_decode(enc, W, b):
    BN = 2048

    def body(enc_ref, w_ref, b_ref, o_ref):
        o_ref[...] = lax.dot_general(
            enc_ref[...], w_ref[...], (((0,), (1,)), ((), ())),
            preferred_element_type=jnp.float32) + b_ref[...]

    return pl.pallas_call(
        body,
        grid=(_N // BN,),
        in_specs=[
            pl.BlockSpec((2 * _N_LEVELS, BN), lambda i: (0, i)),
            pl.BlockSpec((_OUT, 2 * _N_LEVELS), lambda i: (0, 0)),
            pl.BlockSpec((1, _OUT), lambda i: (0, 0)),
        ],
        out_specs=pl.BlockSpec((BN, _OUT), lambda i: (i, 0)),
        out_shape=jax.ShapeDtypeStruct((_N, _OUT), jnp.float32),
    )(enc, W, b[None, :])


def kernel(x, table, shifts, W, b):
    xT = x.T
    tab = table.reshape(_N_LEVELS * _CAP * _N_FEATS)
    scales = jnp.asarray(_scales_np())
    lvc = jnp.concatenate(
        [scales[:, None], shifts, jnp.zeros((_N_LEVELS, 4), jnp.float32)],
        axis=1).reshape(-1)
    enc = _sc_encode(xT, tab, lvc)
    return _tc_decode(enc, W, b)


# speedup vs baseline: 78.9499x; 4.1414x over previous
import functools

import numpy as np
import jax
import jax.numpy as jnp
from jax import lax
from jax.experimental import pallas as pl
from jax.experimental.pallas import tpu as pltpu
from jax.experimental.pallas import tpu_sc as plsc

_POS_DIM = 3
_N_LEVELS = 16
_N_FEATS = 2
_CAP = 2 ** 18
_N = 262144
_OUT = 64

_SFC = [1.0 / float(np.sqrt((i + 1.0) * (i + 2.0))) for i in range(_POS_DIM)]
_PRIMES_I32 = [int(np.int32(np.uint32(p)))
               for p in (2654435761, 805459861, 3674653429)]


def _scales_np():
    l = np.arange(_N_LEVELS, dtype=np.float64)
    g = (1000.0 / 10.0) ** (l / max(_N_LEVELS - 1, 1))
    return (10.0 * g * 1.0).astype(np.float32)


_NC, _NS = 2, 16
_NW = _NC * _NS
_C = 1024
_PPW = _N // _NW
_NCHUNK = _PPW // _C
_G = _C // 16


def _sc_encode(xT, tab, lvc):
    mesh = plsc.VectorSubcoreMesh(
        core_axis_name="c", subcore_axis_name="s",
        num_cores=_NC, num_subcores=_NS)

    @functools.partial(
        pl.kernel,
        out_type=jax.ShapeDtypeStruct((2 * _N_LEVELS, _N), jnp.float32),
        mesh=mesh,
        compiler_params=pltpu.CompilerParams(
            needs_layout_passes=False, use_tc_tiling_on_sc=False),
        scratch_types=[
            pltpu.VMEM((_POS_DIM, _C), jnp.float32),
            pltpu.VMEM((4 * _C,), jnp.int32),
            pltpu.VMEM((4 * _C,), jnp.int32),
            pltpu.VMEM((4 * _C,), jnp.float32),
            pltpu.VMEM((4 * _C,), jnp.float32),
            pltpu.VMEM((4 * _C,), jnp.float32),
            pltpu.VMEM((2 * _N_LEVELS, _C), jnp.float32),
            pltpu.VMEM((8 * _N_LEVELS,), jnp.float32),
            pltpu.SemaphoreType.DMA,
            pltpu.SemaphoreType.DMA,
        ],
    )
    def enc_kernel(xT_hbm, tab_hbm, lvc_hbm, out_hbm,
                   x_v, idx0_v, idx1_v, bary_v, f0_v, f1_v, enc_v, lvc_v,
                   sem0, sem1):
        wid = lax.axis_index("s") * _NC + lax.axis_index("c")
        pltpu.sync_copy(lvc_hbm, lvc_v)
        iota16 = lax.iota(jnp.int32, 16)
        zeros16 = jnp.zeros((16,), jnp.int32)

        def chunk_body(ci, carry):
            base = wid * _PPW + ci * _C
            pltpu.sync_copy(xT_hbm.at[:, pl.ds(base, _C)], x_v)

            def level_body(l, carry2):
                lb = 8 * l
                sc_v = plsc.load_gather(lvc_v, [zeros16 + lb])
                sh0 = plsc.load_gather(lvc_v, [zeros16 + (lb + 1)])
                sh1 = plsc.load_gather(lvc_v, [zeros16 + (lb + 2)])
                sh2 = plsc.load_gather(lvc_v, [zeros16 + (lb + 3)])
                lvl_off = l * (2 * _CAP)

                def group_body(g, carry3):
                    o = g * 16
                    x0 = x_v[0, pl.ds(o, 16)]
                    x1 = x_v[1, pl.ds(o, 16)]
                    x2 = x_v[2, pl.ds(o, 16)]
                    cf0 = (x0 * sc_v + sh0) * _SFC[0]
                    cf1 = (x1 * sc_v + sh1) * _SFC[1]
                    cf2 = (x2 * sc_v + sh2) * _SFC[2]
                    s1 = cf1 + cf2
                    e0 = cf0 + s1
                    e1 = s1 - cf0
                    e2 = cf2 - 2.0 * cf1
                    e3 = -3.0 * cf2
                    ee = [e0, e1, e2, e3]
                    rr = []
                    for e in ee:
                        v = e * 0.25
                        vi = v.astype(jnp.int32)
                        vf = vi.astype(jnp.float32)
                        fl = jnp.where(vf > v, vf - 1.0, vf)
                        ce = jnp.where(vf < v, vf + 1.0, vf)
                        up = ce * 4.0
                        down = fl * 4.0
                        rr.append(jnp.where(up - e < e - down, up, down))
                    ssum = (rr[0] + rr[1]) + (rr[2] + rr[3])
                    rsum = (ssum * 0.25).astype(jnp.int32)
                    dd = [ee[i] - rr[i] for i in range(4)]
                    rank = [rsum, rsum, rsum, rsum]
                    for i in range(4):
                        for j in range(i + 1, 4):
                            c = dd[i] < dd[j]
                            rank[i] = rank[i] + jnp.where(c, 1, 0)
                            rank[j] = rank[j] + jnp.where(c, 0, 1)
                    ri = [r.astype(jnp.int32) for r in rr]
                    dl = []
                    for i in range(4):
                        low = rank[i] < 0
                        high = rank[i] > 3
                        ri[i] = jnp.where(low, ri[i] + 4,
                                          jnp.where(high, ri[i] - 4, ri[i]))
                        rank[i] = jnp.where(low, rank[i] + 4,
                                            jnp.where(high, rank[i] - 4,
                                                      rank[i]))
                        dl.append((ee[i] - ri[i].astype(jnp.float32)) * 0.25)
                    gg = []
                    for r in range(4):
                        acc = jnp.where(rank[0] == r, dl[0], 0.0)
                        for i in range(1, 4):
                            acc = acc + jnp.where(rank[i] == r, dl[i], 0.0)
                        gg.append(acc)
                    bary = [1.0 + gg[3] - gg[0], gg[2] - gg[3],
                            gg[1] - gg[2], gg[0] - gg[1]]
                    for r in range(4):
                        if r == 0:
                            k0, k1, k2 = ri[0], ri[1], ri[2]
                        else:
                            k0 = ri[0] + jnp.where(rank[0] > 3 - r, r - 4, r)
                            k1 = ri[1] + jnp.where(rank[1] > 3 - r, r - 4, r)
                            k2 = ri[2] + jnp.where(rank[2] > 3 - r, r - 4, r)
                        h = ((k0 * _PRIMES_I32[0]) ^ (k1 * _PRIMES_I32[1])
                             ^ (k2 * _PRIMES_I32[2]))
                        p = h & (_CAP - 1)
                        fpos = (lvl_off + ((p >> 7) << 8)) + (p & 127)
                        idx0_v[pl.ds(r * _C + o, 16)] = fpos
                        idx1_v[pl.ds(r * _C + o, 16)] = fpos + 128
                        bary_v[pl.ds(r * _C + o, 16)] = bary[r]
                    return carry3

                lax.fori_loop(0, _G, group_body, 0)
                cp0 = pltpu.async_copy(tab_hbm.at[idx0_v], f0_v, sem0)
                cp1 = pltpu.async_copy(tab_hbm.at[idx1_v], f1_v, sem1)
                cp0.wait()
                cp1.wait()

                def blend_body(g, carry3):
                    o = g * 16
                    acc0 = jnp.zeros((16,), jnp.float32)
                    acc1 = jnp.zeros((16,), jnp.float32)
                    for r in range(4):
                        f0 = f0_v[pl.ds(r * _C + o, 16)]
                        f1 = f1_v[pl.ds(r * _C + o, 16)]
                        br = bary_v[pl.ds(r * _C + o, 16)]
                        acc0 = acc0 + br * f0
                        acc1 = acc1 + br * f1
                    enc_v[2 * l, pl.ds(o, 16)] = acc0
                    enc_v[2 * l + 1, pl.ds(o, 16)] = acc1
                    return carry3

                lax.fori_loop(0, _G, blend_body, 0)
                return carry2

            lax.fori_loop(0, _N_LEVELS, level_body, 0)
            pltpu.sync_copy(enc_v, out_hbm.at[:, pl.ds(base, _C)])
            return carry

        lax.fori_loop(0, _NCHUNK, chunk_body, 0)

    return enc_kernel(xT, tab, lvc)


def _tc_decode(enc, W, b):
    BN = 2048

    def body(enc_ref, w_ref, b_ref, o_ref):
        o_ref[...] = lax.dot_general(
            w_ref[...], enc_ref[...], (((1,), (0,)), ((), ())),
            preferred_element_type=jnp.float32) + b_ref[...]

    outT = pl.pallas_call(
        body,
        grid=(_N // BN,),
        in_specs=[
            pl.BlockSpec((2 * _N_LEVELS, BN), lambda i: (0, i)),
            pl.BlockSpec((_OUT, 2 * _N_LEVELS), lambda i: (0, 0)),
            pl.BlockSpec((_OUT, 1), lambda i: (0, 0)),
        ],
        out_specs=pl.BlockSpec((_OUT, BN), lambda i: (0, i)),
        out_shape=jax.ShapeDtypeStruct((_OUT, _N), jnp.float32),
    )(enc, W, b[:, None])
    return outT.T


def kernel(x, table, shifts, W, b):
    xT = x.T
    tab = (table.reshape(_N_LEVELS, _CAP // 128, 128, _N_FEATS)
           .transpose(0, 1, 3, 2)
           .reshape(_N_LEVELS * _CAP * _N_FEATS))
    scales = jnp.asarray(_scales_np())
    lvc = jnp.concatenate(
        [scales[:, None], shifts, jnp.zeros((_N_LEVELS, 4), jnp.float32)],
        axis=1).reshape(-1)
    enc = _sc_encode(xT, tab, lvc)
    return _tc_decode(enc, W, b)


# speedup vs baseline: 92.8408x; 1.1759x over previous
import functools

import numpy as np
import jax
import jax.numpy as jnp
from jax import lax
from jax.experimental import pallas as pl
from jax.experimental.pallas import tpu as pltpu
from jax.experimental.pallas import tpu_sc as plsc

_POS_DIM = 3
_N_LEVELS = 16
_N_FEATS = 2
_CAP = 2 ** 18
_N = 262144
_OUT = 64

_SFC = [1.0 / float(np.sqrt((i + 1.0) * (i + 2.0))) for i in range(_POS_DIM)]
_PRIMES_I32 = [int(np.int32(np.uint32(p)))
               for p in (2654435761, 805459861, 3674653429)]


def _scales_np():
    l = np.arange(_N_LEVELS, dtype=np.float64)
    g = (1000.0 / 10.0) ** (l / max(_N_LEVELS - 1, 1))
    return (10.0 * g * 1.0).astype(np.float32)


_NC, _NS = 2, 16
_NW = _NC * _NS
_C = 1024
_PPW = _N // _NW
_NCHUNK = _PPW // _C
_G = _C // 16


def _sc_encode(xT, tab, lvc):
    mesh = plsc.VectorSubcoreMesh(
        core_axis_name="c", subcore_axis_name="s",
        num_cores=_NC, num_subcores=_NS)

    @functools.partial(
        pl.kernel,
        out_type=jax.ShapeDtypeStruct((2 * _N_LEVELS, _N), jnp.float32),
        mesh=mesh,
        compiler_params=pltpu.CompilerParams(
            needs_layout_passes=False, use_tc_tiling_on_sc=False),
        scratch_types=[
            pltpu.VMEM((_POS_DIM, _C), jnp.float32),
            pltpu.VMEM((4 * _C,), jnp.int32),
            pltpu.VMEM((4 * _C,), jnp.int32),
            pltpu.VMEM((4 * _C,), jnp.float32),
            pltpu.VMEM((4 * _C,), jnp.float32),
            pltpu.VMEM((4 * _C,), jnp.float32),
            pltpu.VMEM((4 * _C,), jnp.int32),
            pltpu.VMEM((4 * _C,), jnp.int32),
            pltpu.VMEM((4 * _C,), jnp.float32),
            pltpu.VMEM((4 * _C,), jnp.float32),
            pltpu.VMEM((4 * _C,), jnp.float32),
            pltpu.VMEM((2 * _N_LEVELS, _C), jnp.float32),
            pltpu.VMEM((8 * _N_LEVELS,), jnp.float32),
            pltpu.SemaphoreType.DMA,
            pltpu.SemaphoreType.DMA,
            pltpu.SemaphoreType.DMA,
            pltpu.SemaphoreType.DMA,
        ],
    )
    def enc_kernel(xT_hbm, tab_hbm, lvc_hbm, out_hbm,
                   x_v, i0a_v, i1a_v, ba_v, f0a_v, f1a_v,
                   i0b_v, i1b_v, bb_v, f0b_v, f1b_v, enc_v, lvc_v,
                   sa0, sa1, sb0, sb1):
        wid = lax.axis_index("s") * _NC + lax.axis_index("c")
        pltpu.sync_copy(lvc_hbm, lvc_v)
        zeros16 = jnp.zeros((16,), jnp.int32)

        def compute_level(l, idx0_v, idx1_v, bary_v):
            lb = 8 * l
            sc_v = plsc.load_gather(lvc_v, [zeros16 + lb])
            sh0 = plsc.load_gather(lvc_v, [zeros16 + (lb + 1)])
            sh1 = plsc.load_gather(lvc_v, [zeros16 + (lb + 2)])
            sh2 = plsc.load_gather(lvc_v, [zeros16 + (lb + 3)])
            lvl_off = l * (2 * _CAP)

            def group_body(g, carry3):
                o = g * 16
                x0 = x_v[0, pl.ds(o, 16)]
                x1 = x_v[1, pl.ds(o, 16)]
                x2 = x_v[2, pl.ds(o, 16)]
                cf0 = (x0 * sc_v + sh0) * _SFC[0]
                cf1 = (x1 * sc_v + sh1) * _SFC[1]
                cf2 = (x2 * sc_v + sh2) * _SFC[2]
                s1 = cf1 + cf2
                e0 = cf0 + s1
                e1 = s1 - cf0
                e2 = cf2 - 2.0 * cf1
                e3 = -3.0 * cf2
                ee = [e0, e1, e2, e3]
                ci = []
                for e in ee:
                    t = e * 0.25 - 0.5
                    ti = t.astype(jnp.int32)
                    tf = ti.astype(jnp.float32)
                    ci.append(jnp.where(tf < t, ti + 1, ti))
                rsum = (ci[0] + ci[1]) + (ci[2] + ci[3])
                ri = [c * 4 for c in ci]
                dd = [ee[i] - ri[i].astype(jnp.float32) for i in range(4)]
                rank = [rsum, rsum, rsum, rsum]
                for i in range(4):
                    for j in range(i + 1, 4):
                        c = dd[i] < dd[j]
                        rank[i] = rank[i] + jnp.where(c, 1, 0)
                        rank[j] = rank[j] + jnp.where(c, 0, 1)
                dl = []
                for i in range(4):
                    low = rank[i] < 0
                    high = rank[i] > 3
                    ri[i] = jnp.where(low, ri[i] + 4,
                                      jnp.where(high, ri[i] - 4, ri[i]))
                    rank[i] = jnp.where(low, rank[i] + 4,
                                        jnp.where(high, rank[i] - 4,
                                                  rank[i]))
                    dl.append((ee[i] - ri[i].astype(jnp.float32)) * 0.25)
                gg = []
                for r in range(4):
                    acc = jnp.where(rank[0] == r, dl[0], 0.0)
                    for i in range(1, 4):
                        acc = acc + jnp.where(rank[i] == r, dl[i], 0.0)
                    gg.append(acc)
                bary = [1.0 + gg[3] - gg[0], gg[2] - gg[3],
                        gg[1] - gg[2], gg[0] - gg[1]]
                for r in range(4):
                    if r == 0:
                        k0, k1, k2 = ri[0], ri[1], ri[2]
                    else:
                        k0 = ri[0] + jnp.where(rank[0] > 3 - r, r - 4, r)
                        k1 = ri[1] + jnp.where(rank[1] > 3 - r, r - 4, r)
                        k2 = ri[2] + jnp.where(rank[2] > 3 - r, r - 4, r)
                    h = ((k0 * _PRIMES_I32[0]) ^ (k1 * _PRIMES_I32[1])
                         ^ (k2 * _PRIMES_I32[2]))
                    p = h & (_CAP - 1)
                    fpos = (lvl_off + ((p >> 7) << 8)) + (p & 127)
                    idx0_v[pl.ds(r * _C + o, 16)] = fpos
                    idx1_v[pl.ds(r * _C + o, 16)] = fpos + 128
                    bary_v[pl.ds(r * _C + o, 16)] = bary[r]
                return carry3

            lax.fori_loop(0, _G, group_body, 0)

        def blend_level(l, bary_v, f0_v, f1_v):
            def blend_body(g, carry3):
                o = g * 16
                acc0 = jnp.zeros((16,), jnp.float32)
                acc1 = jnp.zeros((16,), jnp.float32)
                for r in range(4):
                    f0 = f0_v[pl.ds(r * _C + o, 16)]
                    f1 = f1_v[pl.ds(r * _C + o, 16)]
                    br = bary_v[pl.ds(r * _C + o, 16)]
                    acc0 = acc0 + br * f0
                    acc1 = acc1 + br * f1
                enc_v[2 * l, pl.ds(o, 16)] = acc0
                enc_v[2 * l + 1, pl.ds(o, 16)] = acc1
                return carry3

            lax.fori_loop(0, _G, blend_body, 0)

        def chunk_body(ci, carry):
            base = wid * _PPW + ci * _C
            pltpu.sync_copy(xT_hbm.at[:, pl.ds(base, _C)], x_v)

            def pair_body(i, carry2):
                l0 = 2 * i
                l1 = 2 * i + 1
                compute_level(l0, i0a_v, i1a_v, ba_v)
                cpa0 = pltpu.async_copy(tab_hbm.at[i0a_v], f0a_v, sa0)
                cpa1 = pltpu.async_copy(tab_hbm.at[i1a_v], f1a_v, sa1)
                compute_level(l1, i0b_v, i1b_v, bb_v)
                cpb0 = pltpu.async_copy(tab_hbm.at[i0b_v], f0b_v, sb0)
                cpb1 = pltpu.async_copy(tab_hbm.at[i1b_v], f1b_v, sb1)
                cpa0.wait()
                cpa1.wait()
                blend_level(l0, ba_v, f0a_v, f1a_v)
                cpb0.wait()
                cpb1.wait()
                blend_level(l1, bb_v, f0b_v, f1b_v)
                return carry2

            lax.fori_loop(0, _N_LEVELS // 2, pair_body, 0)
            pltpu.sync_copy(enc_v, out_hbm.at[:, pl.ds(base, _C)])
            return carry

        lax.fori_loop(0, _NCHUNK, chunk_body, 0)

    return enc_kernel(xT, tab, lvc)


def _tc_decode(enc, W, b):
    BN = 2048

    def body(enc_ref, w_ref, b_ref, o_ref):
        o_ref[...] = lax.dot_general(
            w_ref[...], enc_ref[...], (((1,), (0,)), ((), ())),
            preferred_element_type=jnp.float32) + b_ref[...]

    outT = pl.pallas_call(
        body,
        grid=(_N // BN,),
        in_specs=[
            pl.BlockSpec((2 * _N_LEVELS, BN), lambda i: (0, i)),
            pl.BlockSpec((_OUT, 2 * _N_LEVELS), lambda i: (0, 0)),
            pl.BlockSpec((_OUT, 1), lambda i: (0, 0)),
        ],
        out_specs=pl.BlockSpec((_OUT, BN), lambda i: (0, i)),
        out_shape=jax.ShapeDtypeStruct((_OUT, _N), jnp.float32),
    )(enc, W, b[:, None])
    return outT.T


def kernel(x, table, shifts, W, b):
    xT = x.T
    tab = (table.reshape(_N_LEVELS, _CAP // 128, 128, _N_FEATS)
           .transpose(0, 1, 3, 2)
           .reshape(_N_LEVELS * _CAP * _N_FEATS))
    scales = jnp.asarray(_scales_np())
    lvc = jnp.concatenate(
        [scales[:, None], shifts, jnp.zeros((_N_LEVELS, 4), jnp.float32)],
        axis=1).reshape(-1)
    enc = _sc_encode(xT, tab, lvc)
    return _tc_decode(enc, W, b)
